# baseline (speedup 1.0000x reference)
import functools

import jax
import jax.numpy as jnp
from jax.experimental import pallas as pl

_GRID = 128
_CELLS = _GRID ** 3
_SCENES = 4
_HIDDEN = 16
_N = 262144
_ROWS = _N // 128
_BLK_ROWS = 256
_DECAY = 0.9


def _part1by2(x):
    x = x & jnp.uint32(0x3FF)
    x = (x | (x << 16)) & jnp.uint32(0x30000FF)
    x = (x | (x << 8)) & jnp.uint32(0x300F00F)
    x = (x | (x << 4)) & jnp.uint32(0x30C30C3)
    x = (x | (x << 2)) & jnp.uint32(0x9249249)
    return x


def _sigma_body(coords_ref, code_ref, Wc_ref, w1_ref, b1_ref, w2_ref, b2_ref,
                idx_ref, s01_ref, s23_ref):
    cx = coords_ref[0]
    cy = coords_ref[1]
    cz = coords_ref[2]
    mx = _part1by2(cx.astype(jnp.uint32))
    my = _part1by2(cy.astype(jnp.uint32))
    mz = _part1by2(cz.astype(jnp.uint32))
    idx_ref[...] = (mx | (my << 1) | (mz << 2)).astype(jnp.int32)
    scale = jnp.float32(2.0 / _GRID)
    half = jnp.float32((_GRID - 1) / 2.0)
    fx = (cx.astype(jnp.float32) - half) * scale
    fy = (cy.astype(jnp.float32) - half) * scale
    fz = (cz.astype(jnp.float32) - half) * scale
    cw = jnp.dot(code_ref[...], Wc_ref[...],
                 preferred_element_type=jnp.float32) + b1_ref[...]
    w1 = w1_ref[...]
    w2 = w2_ref[...]
    accs = [jnp.zeros(fx.shape, jnp.float32) for _ in range(_SCENES)]
    for j in range(_HIDDEN):
        base = fx * w1[0, j] + fy * w1[1, j] + fz * w1[2, j]
        for s in range(_SCENES):
            accs[s] = accs[s] + jnp.maximum(base + cw[s, j], 0.0) * w2[j, 0]
    b2v = b2_ref[0, 0]
    sig = [jax.nn.softplus(a + b2v) for a in accs]
    bits = [jnp.uint16(0)] * _SCENES
    for s in range(_SCENES):
        bits[s] = jax.lax.bitcast_convert_type(
            sig[s].astype(jnp.bfloat16), jnp.uint16).astype(jnp.uint32)
    s01_ref[...] = bits[0] | (bits[1] << 16)
    s23_ref[...] = bits[2] | (bits[3] << 16)


@functools.partial(jax.jit, static_argnames=("interpret",))
def _sigma_stage(coords, code, W1, b1, Wc, W2, b2, interpret=False):
    coords3 = coords.T.reshape(3, _ROWS, 128)
    grid = (_ROWS // _BLK_ROWS,)
    full = lambda a: pl.BlockSpec(a.shape, lambda i: tuple(0 for _ in a.shape))
    b1r = b1.reshape(1, _HIDDEN)
    b2r = b2.reshape(1, 1)
    out_shape = [
        jax.ShapeDtypeStruct((_ROWS, 128), jnp.int32),
        jax.ShapeDtypeStruct((_ROWS, 128), jnp.uint32),
        jax.ShapeDtypeStruct((_ROWS, 128), jnp.uint32),
    ]
    idx, s01, s23 = pl.pallas_call(
        _sigma_body,
        grid=grid,
        in_specs=[
            pl.BlockSpec((3, _BLK_ROWS, 128), lambda i: (0, i, 0)),
            full(code), full(Wc), full(W1), full(b1r), full(W2), full(b2r),
        ],
        out_specs=[
            pl.BlockSpec((_BLK_ROWS, 128), lambda i: (i, 0)),
            pl.BlockSpec((_BLK_ROWS, 128), lambda i: (i, 0)),
            pl.BlockSpec((_BLK_ROWS, 128), lambda i: (i, 0)),
        ],
        out_shape=out_shape,
        interpret=interpret,
    )(coords3, code, Wc, W1, b1r, W2, b2r)
    return idx.reshape(-1), s01.reshape(-1), s23.reshape(-1)


def kernel(density_grid, code, W1, b1, Wc, W2, b2, coords):
    idx, s01, s23 = _sigma_stage(coords, code, W1, b1, Wc, W2, b2)
    n_arr = jnp.arange(_N, dtype=jnp.int32)
    winner = jnp.full((_CELLS,), -1, jnp.int32).at[idx].max(n_arr)
    hit = winner >= 0
    w_safe = jnp.where(hit, winner, 0)
    s01g = s01[w_safe]
    s23g = s23[w_safe]
    sig0 = jax.lax.bitcast_convert_type(s01g << 16, jnp.float32)
    sig1 = jax.lax.bitcast_convert_type(s01g & jnp.uint32(0xFFFF0000), jnp.float32)
    sig2 = jax.lax.bitcast_convert_type(s23g << 16, jnp.float32)
    sig3 = jax.lax.bitcast_convert_type(s23g & jnp.uint32(0xFFFF0000), jnp.float32)
    tmp = jnp.stack([sig0, sig1, sig2, sig3], axis=0)
    tmp = jnp.where(hit[None, :], tmp, -1.0)
    valid = (density_grid >= 0) & (tmp >= 0)
    new_grid = jnp.where(valid, jnp.maximum(density_grid * _DECAY, tmp),
                         density_grid)
    mean_density = jnp.mean(jnp.clip(new_grid, 0.0, None))
    return new_grid, mean_density


# speedup vs baseline: 115.3619x; 115.3619x over previous
import functools

import jax
import jax.numpy as jnp
from jax import lax
from jax.experimental import pallas as pl
from jax.experimental.pallas import tpu as pltpu
from jax.experimental.pallas import tpu_sc as plsc

_GRID = 128
_CELLS = _GRID ** 3
_SCENES = 4
_HIDDEN = 16
_N = 262144
_ROWS = _N // 128
_BLK_ROWS = 256
_DECAY = 0.9

_NW = 32
_SPW = _N // _NW
_NB = 64
_BCELLS = _CELLS // _NB
_BSHIFT = 15
_PAD = 4096
_RCHUNK = 2048


def _part1by2(x):
    x = x & jnp.uint32(0x3FF)
    x = (x | (x << 16)) & jnp.uint32(0x30000FF)
    x = (x | (x << 8)) & jnp.uint32(0x300F00F)
    x = (x | (x << 4)) & jnp.uint32(0x30C30C3)
    x = (x | (x << 2)) & jnp.uint32(0x9249249)
    return x


def _sigma_body(coords_ref, code_ref, Wc_ref, w1_ref, b1_ref, w2_ref, b2_ref,
                idx_ref, s01_ref, s23_ref):
    cx = coords_ref[0]
    cy = coords_ref[1]
    cz = coords_ref[2]
    mx = _part1by2(cx.astype(jnp.uint32))
    my = _part1by2(cy.astype(jnp.uint32))
    mz = _part1by2(cz.astype(jnp.uint32))
    idx_ref[...] = (mx | (my << 1) | (mz << 2)).astype(jnp.int32)
    scale = jnp.float32(2.0 / _GRID)
    half = jnp.float32((_GRID - 1) / 2.0)
    fx = (cx.astype(jnp.float32) - half) * scale
    fy = (cy.astype(jnp.float32) - half) * scale
    fz = (cz.astype(jnp.float32) - half) * scale
    cw = jnp.dot(code_ref[...], Wc_ref[...],
                 preferred_element_type=jnp.float32) + b1_ref[...]
    w1 = w1_ref[...]
    w2 = w2_ref[...]
    accs = [jnp.zeros(fx.shape, jnp.float32) for _ in range(_SCENES)]
    for j in range(_HIDDEN):
        base = fx * w1[0, j] + fy * w1[1, j] + fz * w1[2, j]
        for s in range(_SCENES):
            accs[s] = accs[s] + jnp.maximum(base + cw[s, j], 0.0) * w2[j, 0]
    b2v = b2_ref[0, 0]
    sig = [jax.nn.softplus(a + b2v) for a in accs]
    bits = [jax.lax.bitcast_convert_type(
        s.astype(jnp.bfloat16), jnp.uint16).astype(jnp.uint32) for s in sig]
    s01_ref[...] = bits[0] | (bits[1] << 16)
    s23_ref[...] = bits[2] | (bits[3] << 16)


def _sigma_stage(coords, code, W1, b1, Wc, W2, b2):
    coords3 = coords.T.reshape(3, _ROWS, 128)
    grid = (_ROWS // _BLK_ROWS,)
    full = lambda a: pl.BlockSpec(a.shape, lambda i: tuple(0 for _ in a.shape))
    b1r = b1.reshape(1, _HIDDEN)
    b2r = b2.reshape(1, 1)
    out_shape = [
        jax.ShapeDtypeStruct((_ROWS, 128), jnp.int32),
        jax.ShapeDtypeStruct((_ROWS, 128), jnp.uint32),
        jax.ShapeDtypeStruct((_ROWS, 128), jnp.uint32),
    ]
    idx, s01, s23 = pl.pallas_call(
        _sigma_body,
        grid=grid,
        in_specs=[
            pl.BlockSpec((3, _BLK_ROWS, 128), lambda i: (0, i, 0)),
            full(code), full(Wc), full(W1), full(b1r), full(W2), full(b2r),
        ],
        out_specs=[
            pl.BlockSpec((_BLK_ROWS, 128), lambda i: (i, 0)),
            pl.BlockSpec((_BLK_ROWS, 128), lambda i: (i, 0)),
            pl.BlockSpec((_BLK_ROWS, 128), lambda i: (i, 0)),
        ],
        out_shape=out_shape,
    )(coords3, code, Wc, W1, b1r, W2, b2r)
    return idx.reshape(-1), s01.reshape(-1), s23.reshape(-1)


def _wid():
    return lax.axis_index("s") * 2 + lax.axis_index("c")


def _iota16():
    return lax.iota(jnp.int32, 16)


def _scan_count_base():
    cnt0, _ = plsc.scan_count(jnp.zeros((16,), jnp.int32))
    return jnp.min(cnt0)


_SC_MESH = functools.partial(
    plsc.VectorSubcoreMesh, core_axis_name="c", subcore_axis_name="s")
_SC_PARAMS = pltpu.CompilerParams(needs_layout_passes=False)


def _hist_body(idx_hbm, hist_hbm, idx_v, hist_v, sem):
    w = _wid()
    off = _scan_count_base()
    pltpu.async_copy(idx_hbm.at[pl.ds(w * _SPW, _SPW)], idx_v, sem).wait()
    for c in range(4):
        hist_v[pl.ds(c * 16, 16)] = jnp.zeros((16,), jnp.int32)

    def body(i, carry):
        v = idx_v[pl.ds(i * 16, 16)]
        digit = jax.lax.shift_right_logical(v, _BSHIFT)
        cnt, last = plsc.scan_count(digit)
        total = cnt - off + 1
        g = plsc.load_gather(hist_v, [digit])
        plsc.store_scatter(hist_v, [digit], g + total, mask=last)
        return carry

    lax.fori_loop(0, _SPW // 16, body, 0)
    pltpu.async_copy(hist_v, hist_hbm.at[w], sem).wait()


def _hist_stage(idx):
    k = pl.kernel(
        _hist_body,
        out_type=[jax.ShapeDtypeStruct((_NW, _NB), jnp.int32)],
        mesh=_SC_MESH(),
        compiler_params=_SC_PARAMS,
        scratch_types=[
            pltpu.VMEM((_SPW,), jnp.int32),
            pltpu.VMEM((_NB,), jnp.int32),
            pltpu.SemaphoreType.DMA,
        ],
    )
    return k(idx)[0]


def _part_body(idx_hbm, s01_hbm, s23_hbm, hist_hbm,
               bidx_hbm, bs01_hbm, bs23_hbm, tot_hbm,
               idx_v, s01_v, s23_v, hist_v, base_v, tot_v,
               loc_v, v01_v, v23_v, dest_v, sem):
    w = _wid()
    off = _scan_count_base()
    pltpu.async_copy(idx_hbm.at[pl.ds(w * _SPW, _SPW)], idx_v, sem).wait()
    pltpu.async_copy(s01_hbm.at[pl.ds(w * _SPW, _SPW)], s01_v, sem).wait()
    pltpu.async_copy(s23_hbm.at[pl.ds(w * _SPW, _SPW)], s23_v, sem).wait()
    pltpu.async_copy(hist_hbm, hist_v, sem).wait()

    iot = _iota16()
    for c in range(4):
        t = jnp.zeros((16,), jnp.int32)
        p = jnp.zeros((16,), jnp.int32)
        for wp in range(_NW):
            row = hist_v[wp, pl.ds(c * 16, 16)]
            t = t + row
            keep = jnp.full((16,), wp, jnp.int32) < w
            p = p + jnp.where(keep, row, 0)
        tot_v[pl.ds(c * 16, 16)] = t
        base_v[pl.ds(c * 16, 16)] = p
    carry = jnp.zeros((), jnp.int32)
    for c in range(4):
        t = tot_v[pl.ds(c * 16, 16)]
        excl = plsc.cumsum(t) - t + carry
        base_v[pl.ds(c * 16, 16)] = base_v[pl.ds(c * 16, 16)] + excl
        carry = carry + jnp.sum(t)

    @pl.when(w == 0)
    def _():
        pltpu.async_copy(tot_v, tot_hbm, sem).wait()

    def body(j, carry):
        for c in range(8):
            i = j * 8 + c
            v = idx_v[pl.ds(i * 16, 16)]
            digit = jax.lax.shift_right_logical(v, _BSHIFT)
            cnt, last = plsc.scan_count(digit)
            rank = cnt - off
            g = plsc.load_gather(base_v, [digit])
            dest_v[j, pl.ds(c * 16, 16)] = g + rank
            plsc.store_scatter(base_v, [digit], g + rank + 1, mask=last)
            loc_v[j, pl.ds(c * 16, 16)] = v & jnp.int32(_BCELLS - 1)
            v01_v[j, pl.ds(c * 16, 16)] = s01_v[pl.ds(i * 16, 16)]
            v23_v[j, pl.ds(c * 16, 16)] = s23_v[pl.ds(i * 16, 16)]
        return carry

    lax.fori_loop(0, _SPW // 128, body, 0)

    nrows = _SPW // 128
    group = 8
    for g0 in range(0, nrows, group):
        copies = []
        for r in range(g0, g0 + group):
            copies.append(pltpu.async_copy(
                loc_v.at[r], bidx_hbm.at[dest_v.at[r]], sem))
            copies.append(pltpu.async_copy(
                v01_v.at[r], bs01_hbm.at[dest_v.at[r]], sem))
            copies.append(pltpu.async_copy(
                v23_v.at[r], bs23_hbm.at[dest_v.at[r]], sem))
        for cp in copies:
            cp.wait()


def _part_stage(idx, s01, s23, hist):
    k = pl.kernel(
        _part_body,
        out_type=[
            jax.ShapeDtypeStruct((_N + _PAD,), jnp.int32),
            jax.ShapeDtypeStruct((_N + _PAD,), jnp.uint32),
            jax.ShapeDtypeStruct((_N + _PAD,), jnp.uint32),
            jax.ShapeDtypeStruct((_NB,), jnp.int32),
        ],
        mesh=_SC_MESH(),
        compiler_params=_SC_PARAMS,
        scratch_types=[
            pltpu.VMEM((_SPW,), jnp.int32),
            pltpu.VMEM((_SPW,), jnp.uint32),
            pltpu.VMEM((_SPW,), jnp.uint32),
            pltpu.VMEM((_NW, _NB), jnp.int32),
            pltpu.VMEM((_NB,), jnp.int32),
            pltpu.VMEM((_NB,), jnp.int32),
            pltpu.VMEM((_SPW // 128, 128), jnp.int32),
            pltpu.VMEM((_SPW // 128, 128), jnp.uint32),
            pltpu.VMEM((_SPW // 128, 128), jnp.uint32),
            pltpu.VMEM((_SPW // 128, 128), jnp.int32),
            pltpu.SemaphoreType.DMA,
        ],
    )
    return k(idx, s01, s23, hist)


def _apply_body(dg_hbm, bidx_hbm, bs01_hbm, bs23_hbm, tot_hbm,
                out_hbm, part_hbm,
                d_v, tmp_v, rb_v, r01_v, r23_v, tot_v, acc_v, sem):
    w = _wid()
    iot = _iota16()
    pltpu.async_copy(tot_hbm, tot_v, sem).wait()
    acc = jnp.zeros((16,), jnp.float32)
    for s in range(_SCENES):
        for h in range(2):
            b = w * 2 + h
            start = jnp.zeros((), jnp.int32)
            count = jnp.zeros((), jnp.int32)
            for c in range(4):
                t = tot_v[pl.ds(c * 16, 16)]
                lane = iot + c * 16
                start = start + jnp.sum(jnp.where(lane < b, t, 0))
                count = count + jnp.sum(jnp.where(lane == b, t, 0))
            end = start + count
            astart = pl.multiple_of(start & jnp.int32(~7), 8)
            nchunks = (end - astart + (_RCHUNK - 1)) // _RCHUNK

            pltpu.async_copy(
                dg_hbm.at[s, pl.ds(b * _BCELLS, _BCELLS)], d_v, sem).wait()

            def fill(i, carry):
                for c in range(8):
                    tmp_v[pl.ds((i * 8 + c) * 16, 16)] = jnp.full(
                        (16,), -1.0, jnp.float32)
                return carry

            lax.fori_loop(0, _BCELLS // 128, fill, 0)

            def apply_chunk(ci, carry):
                cbase = astart + ci * _RCHUNK
                pltpu.async_copy(
                    bidx_hbm.at[pl.ds(cbase, _RCHUNK)], rb_v, sem).wait()
                if s < 2:
                    pltpu.async_copy(
                        bs01_hbm.at[pl.ds(cbase, _RCHUNK)], r01_v, sem).wait()
                else:
                    pltpu.async_copy(
                        bs23_hbm.at[pl.ds(cbase, _RCHUNK)], r23_v, sem).wait()

                def inner(vi, c2):
                    for c in range(4):
                        k = vi * 4 + c
                        pos = cbase + k * 16 + iot
                        m = (pos >= start) & (pos < end)
                        loc = rb_v[pl.ds(k * 16, 16)] & jnp.int32(_BCELLS - 1)
                        word = (r01_v if s < 2 else r23_v)[pl.ds(k * 16, 16)]
                        if s % 2 == 0:
                            bits = word << 16
                        else:
                            bits = word & jnp.uint32(0xFFFF0000)
                        sigv = plsc.bitcast(bits, jnp.float32)
                        plsc.store_scatter(tmp_v, [loc], sigv, mask=m)
                    return c2

                lax.fori_loop(0, _RCHUNK // 64, inner, 0)
                return carry

            lax.fori_loop(0, nchunks, apply_chunk, 0)

            def merge(i, a):
                for c in range(8):
                    sl = pl.ds((i * 8 + c) * 16, 16)
                    d = d_v[sl]
                    t = tmp_v[sl]
                    o = jnp.where(t >= 0.0,
                                  jnp.maximum(d * jnp.float32(_DECAY), t), d)
                    d_v[sl] = o
                    a = a + o
                return a

            acc = lax.fori_loop(0, _BCELLS // 128, merge, acc)
            pltpu.async_copy(
                d_v, out_hbm.at[s, pl.ds(b * _BCELLS, _BCELLS)], sem).wait()
    acc_v[...] = acc
    pltpu.async_copy(acc_v, part_hbm.at[w], sem).wait()


def _apply_stage(density_grid, bidx, bs01, bs23, tot):
    k = pl.kernel(
        _apply_body,
        out_type=[
            jax.ShapeDtypeStruct((_SCENES, _CELLS), jnp.float32),
            jax.ShapeDtypeStruct((_NW, 16), jnp.float32),
        ],
        mesh=_SC_MESH(),
        compiler_params=_SC_PARAMS,
        scratch_types=[
            pltpu.VMEM((_BCELLS,), jnp.float32),
            pltpu.VMEM((_BCELLS,), jnp.float32),
            pltpu.VMEM((_RCHUNK,), jnp.int32),
            pltpu.VMEM((_RCHUNK,), jnp.uint32),
            pltpu.VMEM((_RCHUNK,), jnp.uint32),
            pltpu.VMEM((_NB,), jnp.int32),
            pltpu.VMEM((16,), jnp.float32),
            pltpu.SemaphoreType.DMA,
        ],
    )
    return k(density_grid, bidx, bs01, bs23, tot)


def kernel(density_grid, code, W1, b1, Wc, W2, b2, coords):
    idx, s01, s23 = _sigma_stage(coords, code, W1, b1, Wc, W2, b2)
    hist = _hist_stage(idx)
    bidx, bs01, bs23, tot = _part_stage(idx, s01, s23, hist)
    new_grid, partials = _apply_stage(density_grid, bidx, bs01, bs23, tot)
    mean_density = jnp.sum(partials) / jnp.float32(_SCENES * _CELLS)
    return new_grid, mean_density


# speedup vs baseline: 851.9586x; 7.3851x over previous
import functools

import jax
import jax.numpy as jnp
from jax import lax
from jax.experimental import pallas as pl
from jax.experimental.pallas import tpu as pltpu
from jax.experimental.pallas import tpu_sc as plsc

_GRID = 128
_CELLS = _GRID ** 3
_SCENES = 4
_HIDDEN = 16
_N = 262144
_ROWS = _N // 128
_BLK_ROWS = 256
_DECAY = 0.9

_NW = 32
_SPW = _N // _NW
_NB = 64
_BCELLS = _CELLS // _NB
_BSHIFT = 15
_PAD = 1024
_SLOT = 256


def _part1by2(x):
    x = x & jnp.uint32(0x3FF)
    x = (x | (x << 16)) & jnp.uint32(0x30000FF)
    x = (x | (x << 8)) & jnp.uint32(0x300F00F)
    x = (x | (x << 4)) & jnp.uint32(0x30C30C3)
    x = (x | (x << 2)) & jnp.uint32(0x9249249)
    return x


def _sigma_body(coords_ref, code_ref, Wc_ref, w1_ref, b1_ref, w2_ref, b2_ref,
                idx_ref, s01_ref, s23_ref):
    cx = coords_ref[0]
    cy = coords_ref[1]
    cz = coords_ref[2]
    mx = _part1by2(cx.astype(jnp.uint32))
    my = _part1by2(cy.astype(jnp.uint32))
    mz = _part1by2(cz.astype(jnp.uint32))
    idx_ref[...] = (mx | (my << 1) | (mz << 2)).astype(jnp.int32)
    scale = jnp.float32(2.0 / _GRID)
    half = jnp.float32((_GRID - 1) / 2.0)
    fx = (cx.astype(jnp.float32) - half) * scale
    fy = (cy.astype(jnp.float32) - half) * scale
    fz = (cz.astype(jnp.float32) - half) * scale
    cw = jnp.dot(code_ref[...], Wc_ref[...],
                 preferred_element_type=jnp.float32) + b1_ref[...]
    w1 = w1_ref[...]
    w2 = w2_ref[...]
    accs = [jnp.zeros(fx.shape, jnp.float32) for _ in range(_SCENES)]
    for j in range(_HIDDEN):
        base = fx * w1[0, j] + fy * w1[1, j] + fz * w1[2, j]
        for s in range(_SCENES):
            accs[s] = accs[s] + jnp.maximum(base + cw[s, j], 0.0) * w2[j, 0]
    b2v = b2_ref[0, 0]
    sig = [jax.nn.softplus(a + b2v) for a in accs]
    bits = [jax.lax.bitcast_convert_type(
        s.astype(jnp.bfloat16), jnp.uint16).astype(jnp.uint32) for s in sig]
    s01_ref[...] = bits[0] | (bits[1] << 16)
    s23_ref[...] = bits[2] | (bits[3] << 16)


def _sigma_stage(coords, code, W1, b1, Wc, W2, b2):
    coords3 = coords.T.reshape(3, _ROWS, 128)
    grid = (_ROWS // _BLK_ROWS,)
    full = lambda a: pl.BlockSpec(a.shape, lambda i: tuple(0 for _ in a.shape))
    b1r = b1.reshape(1, _HIDDEN)
    b2r = b2.reshape(1, 1)
    out_shape = [
        jax.ShapeDtypeStruct((_ROWS, 128), jnp.int32),
        jax.ShapeDtypeStruct((_ROWS, 128), jnp.uint32),
        jax.ShapeDtypeStruct((_ROWS, 128), jnp.uint32),
    ]
    idx, s01, s23 = pl.pallas_call(
        _sigma_body,
        grid=grid,
        in_specs=[
            pl.BlockSpec((3, _BLK_ROWS, 128), lambda i: (0, i, 0)),
            full(code), full(Wc), full(W1), full(b1r), full(W2), full(b2r),
        ],
        out_specs=[
            pl.BlockSpec((_BLK_ROWS, 128), lambda i: (i, 0)),
            pl.BlockSpec((_BLK_ROWS, 128), lambda i: (i, 0)),
            pl.BlockSpec((_BLK_ROWS, 128), lambda i: (i, 0)),
        ],
        out_shape=out_shape,
    )(coords3, code, Wc, W1, b1r, W2, b2r)
    return idx.reshape(-1), s01.reshape(-1), s23.reshape(-1)


def _wid():
    return lax.axis_index("s") * 2 + lax.axis_index("c")


def _iota16():
    return lax.iota(jnp.int32, 16)


def _scan_count_base():
    cnt0, _ = plsc.scan_count(jnp.zeros((16,), jnp.int32))
    return jnp.min(cnt0)


_SC_MESH = functools.partial(
    plsc.VectorSubcoreMesh, core_axis_name="c", subcore_axis_name="s")
_SC_PARAMS = pltpu.CompilerParams(needs_layout_passes=False)


def _route_body(idx_hbm, s01_hbm, s23_hbm,
                rloc_hbm, r01_hbm, r23_hbm, hist_hbm,
                idx_v, s01_v, s23_v, hist_v, lbase_v,
                loc_v, o01_v, o23_v, sem):
    w = _wid()
    off = _scan_count_base()
    cp1 = pltpu.async_copy(idx_hbm.at[pl.ds(w * _SPW, _SPW)], idx_v, sem)
    cp2 = pltpu.async_copy(s01_hbm.at[pl.ds(w * _SPW, _SPW)], s01_v, sem)
    cp3 = pltpu.async_copy(s23_hbm.at[pl.ds(w * _SPW, _SPW)], s23_v, sem)
    cp1.wait()
    for c in range(4):
        hist_v[pl.ds(c * 16, 16)] = jnp.zeros((16,), jnp.int32)

    def hbody(i, carry):
        v = idx_v[pl.ds(i * 16, 16)]
        digit = jax.lax.shift_right_logical(v, _BSHIFT)
        cnt, last = plsc.scan_count(digit)
        g = plsc.load_gather(hist_v, [digit])
        plsc.store_scatter(hist_v, [digit], g + cnt - off + 1, mask=last)
        return carry

    lax.fori_loop(0, _SPW // 16, hbody, 0)
    cph = pltpu.async_copy(hist_v, hist_hbm.at[w], sem)

    carry = jnp.zeros((), jnp.int32)
    for c in range(4):
        t = hist_v[pl.ds(c * 16, 16)]
        lbase_v[pl.ds(c * 16, 16)] = plsc.cumsum(t) - t + carry
        carry = carry + jnp.sum(t)

    cp2.wait()
    cp3.wait()

    def pbody(i, carry):
        v = idx_v[pl.ds(i * 16, 16)]
        digit = jax.lax.shift_right_logical(v, _BSHIFT)
        cnt, last = plsc.scan_count(digit)
        rank = cnt - off
        g = plsc.load_gather(lbase_v, [digit])
        dest = g + rank
        plsc.store_scatter(lbase_v, [digit], dest + 1, mask=last)
        plsc.store_scatter(loc_v, [dest], v & jnp.int32(_BCELLS - 1))
        plsc.store_scatter(o01_v, [dest],
                           plsc.bitcast(s01_v[pl.ds(i * 16, 16)], jnp.int32))
        plsc.store_scatter(o23_v, [dest],
                           plsc.bitcast(s23_v[pl.ds(i * 16, 16)], jnp.int32))
        return carry

    lax.fori_loop(0, _SPW // 16, pbody, 0)

    co1 = pltpu.async_copy(loc_v, rloc_hbm.at[pl.ds(w * _SPW, _SPW)], sem)
    co2 = pltpu.async_copy(o01_v, r01_hbm.at[pl.ds(w * _SPW, _SPW)], sem)
    co3 = pltpu.async_copy(o23_v, r23_hbm.at[pl.ds(w * _SPW, _SPW)], sem)
    cph.wait()
    co1.wait()
    co2.wait()
    co3.wait()


def _route_stage(idx, s01, s23):
    k = pl.kernel(
        _route_body,
        out_type=[
            jax.ShapeDtypeStruct((_N + _PAD,), jnp.int32),
            jax.ShapeDtypeStruct((_N + _PAD,), jnp.int32),
            jax.ShapeDtypeStruct((_N + _PAD,), jnp.int32),
            jax.ShapeDtypeStruct((_NW, _NB), jnp.int32),
        ],
        mesh=_SC_MESH(),
        compiler_params=_SC_PARAMS,
        scratch_types=[
            pltpu.VMEM((_SPW,), jnp.int32),
            pltpu.VMEM((_SPW,), jnp.uint32),
            pltpu.VMEM((_SPW,), jnp.uint32),
            pltpu.VMEM((_NB,), jnp.int32),
            pltpu.VMEM((_NB,), jnp.int32),
            pltpu.VMEM((_SPW,), jnp.int32),
            pltpu.VMEM((_SPW,), jnp.int32),
            pltpu.VMEM((_SPW,), jnp.int32),
            pltpu.SemaphoreType.DMA,
        ],
    )
    return k(idx, s01, s23)


def _extract(row_ref, wp, col):
    iot = _iota16()
    acc = jnp.zeros((), jnp.int32)
    for c in range(4):
        t = row_ref[wp, pl.ds(c * 16, 16)]
        acc = acc + jnp.sum(jnp.where(iot + c * 16 == col, t, 0))
    return acc


def _apply_body(dg_hbm, rloc_hbm, r01_hbm, r23_hbm, hist_hbm,
                out_hbm, part_hbm,
                d0_v, d1_v, tmp_v, sl_v, s01_v, s23_v, hist_v,
                meta_s, acc_v, sem):
    w = _wid()
    iot = _iota16()
    pltpu.async_copy(hist_hbm, hist_v, sem).wait()

    def packrow(wp, carry):
        c0 = jnp.zeros((), jnp.int32)
        for c in range(4):
            t = hist_v[wp, pl.ds(c * 16, 16)]
            excl = plsc.cumsum(t) - t + c0
            c0 = c0 + jnp.sum(t)
            hist_v[wp, pl.ds(c * 16, 16)] = (excl << 13) | t
        return carry

    lax.fori_loop(0, _NW, packrow, 0)

    acc = jnp.zeros((16,), jnp.float32)
    pending_out = [None, None]
    for h in range(2):
        b = w * 2 + h

        def stage(wp, carry):
            packed = _extract(hist_v, wp, b)
            glen = packed & jnp.int32(8191)
            gstart = wp * _SPW + (packed >> 13)
            astart = pl.multiple_of(gstart & jnp.int32(~7), 8)
            meta_s[2 * wp] = glen
            meta_s[2 * wp + 1] = gstart - astart
            for k in range(_SLOT // 128):
                src = pl.ds(astart + k * 128, 128)
                dst = pl.ds(wp * _SLOT + k * 128, 128)
                pltpu.async_copy(rloc_hbm.at[src], sl_v.at[dst], sem)
                pltpu.async_copy(r01_hbm.at[src], s01_v.at[dst], sem)
                pltpu.async_copy(r23_hbm.at[src], s23_v.at[dst], sem)
            return carry

        lax.fori_loop(0, _NW, stage, 0)

        def drain(i, carry):
            pltpu.make_async_copy(
                rloc_hbm.at[pl.ds(0, 128)], sl_v.at[pl.ds(0, 128)],
                sem).wait()
            return carry

        lax.fori_loop(0, _NW * (_SLOT // 128) * 3, drain, 0)

        for s in range(_SCENES):
            if pending_out[s % 2] is not None:
                pending_out[s % 2].wait()
                pending_out[s % 2] = None
            dbuf = d0_v if s % 2 == 0 else d1_v
            cpd = pltpu.async_copy(
                dg_hbm.at[s, pl.ds(b * _BCELLS, _BCELLS)], dbuf, sem)

            def fill(i, carry):
                for c in range(8):
                    tmp_v[pl.ds((i * 8 + c) * 16, 16)] = jnp.full(
                        (16,), -1.0, jnp.float32)
                return carry

            lax.fori_loop(0, _BCELLS // 128, fill, 0)

            def segs(wp, carry, s=s):
                shift = meta_s[2 * wp + 1]
                endp = shift + meta_s[2 * wp]

                def seg(vi, c2):
                    pos = vi * 16 + iot
                    m = (pos >= shift) & (pos < endp)
                    sl = pl.ds(wp * _SLOT + vi * 16, 16)
                    loc = sl_v[sl] & jnp.int32(_BCELLS - 1)
                    word = (s01_v if s < 2 else s23_v)[sl]
                    if s % 2 == 0:
                        bits = jax.lax.shift_left(word, 16)
                    else:
                        bits = word & jnp.int32(-65536)
                    sigv = plsc.bitcast(bits, jnp.float32)
                    plsc.store_scatter(tmp_v, [loc], sigv, mask=m)
                    return c2

                lax.fori_loop(0, (endp + 15) // 16, seg, 0)
                return carry

            lax.fori_loop(0, _NW, segs, 0)

            cpd.wait()

            def merge(i, a):
                for c in range(8):
                    slc = pl.ds((i * 8 + c) * 16, 16)
                    d = dbuf[slc]
                    t = tmp_v[slc]
                    o = jnp.where(t >= 0.0,
                                  jnp.maximum(d * jnp.float32(_DECAY), t), d)
                    dbuf[slc] = o
                    a = a + o
                return a

            acc = lax.fori_loop(0, _BCELLS // 128, merge, acc)
            pending_out[s % 2] = pltpu.async_copy(
                dbuf, out_hbm.at[s, pl.ds(b * _BCELLS, _BCELLS)], sem)
    for cp in pending_out:
        if cp is not None:
            cp.wait()
    acc_v[...] = acc
    pltpu.async_copy(acc_v, part_hbm.at[w], sem).wait()


def _apply_stage(density_grid, rloc, r01, r23, hist):
    k = pl.kernel(
        _apply_body,
        out_type=[
            jax.ShapeDtypeStruct((_SCENES, _CELLS), jnp.float32),
            jax.ShapeDtypeStruct((_NW, 16), jnp.float32),
        ],
        mesh=_SC_MESH(),
        compiler_params=_SC_PARAMS,
        scratch_types=[
            pltpu.VMEM((_BCELLS,), jnp.float32),
            pltpu.VMEM((_BCELLS,), jnp.float32),
            pltpu.VMEM((_BCELLS,), jnp.float32),
            pltpu.VMEM((_NW * _SLOT,), jnp.int32),
            pltpu.VMEM((_NW * _SLOT,), jnp.int32),
            pltpu.VMEM((_NW * _SLOT,), jnp.int32),
            pltpu.VMEM((_NW, _NB), jnp.int32),
            pltpu.SMEM((2 * _NW,), jnp.int32),
            pltpu.VMEM((16,), jnp.float32),
            pltpu.SemaphoreType.DMA,
        ],
    )
    return k(density_grid, rloc, r01, r23, hist)


def kernel(density_grid, code, W1, b1, Wc, W2, b2, coords):
    idx, s01, s23 = _sigma_stage(coords, code, W1, b1, Wc, W2, b2)
    rloc, r01, r23, hist = _route_stage(idx, s01, s23)
    new_grid, partials = _apply_stage(density_grid, rloc, r01, r23, hist)
    mean_density = jnp.sum(partials) / jnp.float32(_SCENES * _CELLS)
    return new_grid, mean_density


# speedup vs baseline: 941.7067x; 1.1053x over previous
import functools

import jax
import jax.numpy as jnp
from jax import lax
from jax.experimental import pallas as pl
from jax.experimental.pallas import tpu as pltpu
from jax.experimental.pallas import tpu_sc as plsc

_GRID = 128
_CELLS = _GRID ** 3
_SCENES = 4
_HIDDEN = 16
_N = 262144
_ROWS = _N // 128
_BLK_ROWS = 256
_DECAY = 0.9

_NW = 32
_SPW = _N // _NW
_NB = 64
_BCELLS = _CELLS // _NB
_BSHIFT = 15
_PAD = 1024
_SLOT = 256


def _part1by2(x):
    x = x & jnp.uint32(0x3FF)
    x = (x | (x << 16)) & jnp.uint32(0x30000FF)
    x = (x | (x << 8)) & jnp.uint32(0x300F00F)
    x = (x | (x << 4)) & jnp.uint32(0x30C30C3)
    x = (x | (x << 2)) & jnp.uint32(0x9249249)
    return x


def _sigma_body(coords_ref, code_ref, Wc_ref, w1_ref, b1_ref, w2_ref, b2_ref,
                idx_ref, s01_ref, s23_ref):
    cx = coords_ref[0]
    cy = coords_ref[1]
    cz = coords_ref[2]
    mx = _part1by2(cx.astype(jnp.uint32))
    my = _part1by2(cy.astype(jnp.uint32))
    mz = _part1by2(cz.astype(jnp.uint32))
    idx_ref[...] = (mx | (my << 1) | (mz << 2)).astype(jnp.int32)
    scale = jnp.float32(2.0 / _GRID)
    half = jnp.float32((_GRID - 1) / 2.0)
    fx = (cx.astype(jnp.float32) - half) * scale
    fy = (cy.astype(jnp.float32) - half) * scale
    fz = (cz.astype(jnp.float32) - half) * scale
    cw = jnp.dot(code_ref[...], Wc_ref[...],
                 preferred_element_type=jnp.float32) + b1_ref[...]
    w1 = w1_ref[...]
    w2 = w2_ref[...]
    accs = [jnp.zeros(fx.shape, jnp.float32) for _ in range(_SCENES)]
    for j in range(_HIDDEN):
        base = fx * w1[0, j] + fy * w1[1, j] + fz * w1[2, j]
        for s in range(_SCENES):
            accs[s] = accs[s] + jnp.maximum(base + cw[s, j], 0.0) * w2[j, 0]
    b2v = b2_ref[0, 0]
    sig = [jax.nn.softplus(a + b2v) for a in accs]
    bits = [jax.lax.bitcast_convert_type(
        s.astype(jnp.bfloat16), jnp.uint16).astype(jnp.uint32) for s in sig]
    s01_ref[...] = bits[0] | (bits[1] << 16)
    s23_ref[...] = bits[2] | (bits[3] << 16)


def _sigma_stage(coords, code, W1, b1, Wc, W2, b2):
    coords3 = coords.T.reshape(3, _ROWS, 128)
    grid = (_ROWS // _BLK_ROWS,)
    full = lambda a: pl.BlockSpec(a.shape, lambda i: tuple(0 for _ in a.shape))
    b1r = b1.reshape(1, _HIDDEN)
    b2r = b2.reshape(1, 1)
    out_shape = [
        jax.ShapeDtypeStruct((_ROWS, 128), jnp.int32),
        jax.ShapeDtypeStruct((_ROWS, 128), jnp.uint32),
        jax.ShapeDtypeStruct((_ROWS, 128), jnp.uint32),
    ]
    idx, s01, s23 = pl.pallas_call(
        _sigma_body,
        grid=grid,
        in_specs=[
            pl.BlockSpec((3, _BLK_ROWS, 128), lambda i: (0, i, 0)),
            full(code), full(Wc), full(W1), full(b1r), full(W2), full(b2r),
        ],
        out_specs=[
            pl.BlockSpec((_BLK_ROWS, 128), lambda i: (i, 0)),
            pl.BlockSpec((_BLK_ROWS, 128), lambda i: (i, 0)),
            pl.BlockSpec((_BLK_ROWS, 128), lambda i: (i, 0)),
        ],
        out_shape=out_shape,
    )(coords3, code, Wc, W1, b1r, W2, b2r)
    return idx.reshape(-1), s01.reshape(-1), s23.reshape(-1)


def _wid():
    return lax.axis_index("s") * 2 + lax.axis_index("c")


def _iota16():
    return lax.iota(jnp.int32, 16)


def _scan_count_base():
    cnt0, _ = plsc.scan_count(jnp.zeros((16,), jnp.int32))
    return jnp.min(cnt0)


_SC_MESH = functools.partial(
    plsc.VectorSubcoreMesh, core_axis_name="c", subcore_axis_name="s")
_SC_PARAMS = pltpu.CompilerParams(needs_layout_passes=False)


def _route_body(idx_hbm, s01_hbm, s23_hbm,
                rloc_hbm, r01_hbm, r23_hbm, hist_hbm,
                idx_v, s01_v, s23_v, hist_v, lbase_v,
                loc_v, o01_v, o23_v, sem):
    w = _wid()
    off = _scan_count_base()
    cp1 = pltpu.async_copy(idx_hbm.at[pl.ds(w * _SPW, _SPW)], idx_v, sem)
    cp2 = pltpu.async_copy(s01_hbm.at[pl.ds(w * _SPW, _SPW)], s01_v, sem)
    cp3 = pltpu.async_copy(s23_hbm.at[pl.ds(w * _SPW, _SPW)], s23_v, sem)
    cp1.wait()
    for c in range(4):
        hist_v[pl.ds(c * 16, 16)] = jnp.zeros((16,), jnp.int32)

    def hbody(i, carry):
        v = idx_v[pl.ds(i * 16, 16)]
        digit = jax.lax.shift_right_logical(v, _BSHIFT)
        cnt, last = plsc.scan_count(digit)
        g = plsc.load_gather(hist_v, [digit])
        plsc.store_scatter(hist_v, [digit], g + cnt - off + 1, mask=last)
        return carry

    lax.fori_loop(0, _SPW // 16, hbody, 0)
    cph = pltpu.async_copy(hist_v, hist_hbm.at[w], sem)

    carry = jnp.zeros((), jnp.int32)
    for c in range(4):
        t = hist_v[pl.ds(c * 16, 16)]
        lbase_v[pl.ds(c * 16, 16)] = plsc.cumsum(t) - t + carry
        carry = carry + jnp.sum(t)

    cp2.wait()
    cp3.wait()

    def pbody(i, carry):
        v = idx_v[pl.ds(i * 16, 16)]
        digit = jax.lax.shift_right_logical(v, _BSHIFT)
        cnt, last = plsc.scan_count(digit)
        rank = cnt - off
        g = plsc.load_gather(lbase_v, [digit])
        dest = g + rank
        plsc.store_scatter(lbase_v, [digit], dest + 1, mask=last)
        plsc.store_scatter(loc_v, [dest], v & jnp.int32(_BCELLS - 1))
        plsc.store_scatter(o01_v, [dest],
                           plsc.bitcast(s01_v[pl.ds(i * 16, 16)], jnp.int32))
        plsc.store_scatter(o23_v, [dest],
                           plsc.bitcast(s23_v[pl.ds(i * 16, 16)], jnp.int32))
        return carry

    lax.fori_loop(0, _SPW // 16, pbody, 0)

    co1 = pltpu.async_copy(loc_v, rloc_hbm.at[pl.ds(w * _SPW, _SPW)], sem)
    co2 = pltpu.async_copy(o01_v, r01_hbm.at[pl.ds(w * _SPW, _SPW)], sem)
    co3 = pltpu.async_copy(o23_v, r23_hbm.at[pl.ds(w * _SPW, _SPW)], sem)
    cph.wait()
    co1.wait()
    co2.wait()
    co3.wait()


def _route_stage(idx, s01, s23):
    k = pl.kernel(
        _route_body,
        out_type=[
            jax.ShapeDtypeStruct((_N + _PAD,), jnp.int32),
            jax.ShapeDtypeStruct((_N + _PAD,), jnp.int32),
            jax.ShapeDtypeStruct((_N + _PAD,), jnp.int32),
            jax.ShapeDtypeStruct((_NW, _NB), jnp.int32),
        ],
        mesh=_SC_MESH(),
        compiler_params=_SC_PARAMS,
        scratch_types=[
            pltpu.VMEM((_SPW,), jnp.int32),
            pltpu.VMEM((_SPW,), jnp.uint32),
            pltpu.VMEM((_SPW,), jnp.uint32),
            pltpu.VMEM((_NB,), jnp.int32),
            pltpu.VMEM((_NB,), jnp.int32),
            pltpu.VMEM((_SPW,), jnp.int32),
            pltpu.VMEM((_SPW,), jnp.int32),
            pltpu.VMEM((_SPW,), jnp.int32),
            pltpu.SemaphoreType.DMA,
        ],
    )
    return k(idx, s01, s23)


def _extract(row_ref, wp, col):
    iot = _iota16()
    acc = jnp.zeros((), jnp.int32)
    for c in range(4):
        t = row_ref[wp, pl.ds(c * 16, 16)]
        acc = acc + jnp.sum(jnp.where(iot + c * 16 == col, t, 0))
    return acc


def _apply_body(dg_hbm, rloc_hbm, r01_hbm, r23_hbm, hist_hbm,
                out_hbm, part_hbm,
                d0_v, d1_v, tmp_v, sl_v, s01_v, s23_v, hist_v,
                meta_s, acc_v, sem):
    w = _wid()
    iot = _iota16()
    pltpu.async_copy(hist_hbm, hist_v, sem).wait()

    def packrow(wp, carry):
        c0 = jnp.zeros((), jnp.int32)
        for c in range(4):
            t = hist_v[wp, pl.ds(c * 16, 16)]
            excl = plsc.cumsum(t) - t + c0
            c0 = c0 + jnp.sum(t)
            hist_v[wp, pl.ds(c * 16, 16)] = (excl << 13) | t
        return carry

    lax.fori_loop(0, _NW, packrow, 0)

    def fill0(i, carry):
        for c in range(8):
            tmp_v[pl.ds((i * 8 + c) * 16, 16)] = jnp.zeros((16,), jnp.int32)
        return carry

    lax.fori_loop(0, _BCELLS // 128, fill0, 0)

    acc = jnp.zeros((16,), jnp.float32)
    pending_out = [None, None]
    for h in range(2):
        b = w * 2 + h

        def stage(wp, carry):
            packed = _extract(hist_v, wp, b)
            glen = packed & jnp.int32(8191)
            gstart = wp * _SPW + (packed >> 13)
            astart = pl.multiple_of(gstart & jnp.int32(~7), 8)
            meta_s[2 * wp] = glen
            meta_s[2 * wp + 1] = gstart - astart
            for k in range(_SLOT // 128):
                src = pl.ds(astart + k * 128, 128)
                dst = pl.ds(wp * _SLOT + k * 128, 128)
                pltpu.async_copy(rloc_hbm.at[src], sl_v.at[dst], sem)
                pltpu.async_copy(r01_hbm.at[src], s01_v.at[dst], sem)
                pltpu.async_copy(r23_hbm.at[src], s23_v.at[dst], sem)
            return carry

        lax.fori_loop(0, _NW, stage, 0)

        def drain(i, carry):
            pltpu.make_async_copy(
                rloc_hbm.at[pl.ds(0, 128)], sl_v.at[pl.ds(0, 128)],
                sem).wait()
            return carry

        lax.fori_loop(0, _NW * (_SLOT // 128) * 3, drain, 0)

        def scatter_pass(mode):
            def segs(wp, carry):
                shift = meta_s[2 * wp + 1]
                endp = shift + meta_s[2 * wp]

                def seg(vi, c2):
                    pos = vi * 16 + iot
                    m = (pos >= shift) & (pos < endp)
                    sl = pl.ds(wp * _SLOT + vi * 16, 16)
                    loc = sl_v[sl] & jnp.int32(_BCELLS - 1)
                    if mode == 0:
                        word = s01_v[sl]
                    elif mode == 1:
                        word = s23_v[sl]
                    else:
                        word = jnp.zeros((16,), jnp.int32)
                    plsc.store_scatter(tmp_v, [loc], word, mask=m)
                    return c2

                lax.fori_loop(0, (endp + 15) // 16, seg, 0)
                return carry

            lax.fori_loop(0, _NW, segs, 0)

        for s in range(_SCENES):
            if pending_out[s % 2] is not None:
                pending_out[s % 2].wait()
                pending_out[s % 2] = None
            dbuf = d0_v if s % 2 == 0 else d1_v
            cpd = pltpu.async_copy(
                dg_hbm.at[s, pl.ds(b * _BCELLS, _BCELLS)], dbuf, sem)
            if s % 2 == 0:
                scatter_pass(s // 2)
            cpd.wait()

            def merge(i, a, s=s):
                for c in range(8):
                    slc = pl.ds((i * 8 + c) * 16, 16)
                    d = dbuf[slc]
                    word = tmp_v[slc]
                    if s % 2 == 0:
                        bits = jax.lax.shift_left(word, 16)
                    else:
                        bits = word & jnp.int32(-65536)
                    t = plsc.bitcast(bits, jnp.float32)
                    o = jnp.where(bits != 0,
                                  jnp.maximum(d * jnp.float32(_DECAY), t), d)
                    dbuf[slc] = o
                    a = a + o
                return a

            acc = lax.fori_loop(0, _BCELLS // 128, merge, acc)
            pending_out[s % 2] = pltpu.async_copy(
                dbuf, out_hbm.at[s, pl.ds(b * _BCELLS, _BCELLS)], sem)
        if h == 0:
            scatter_pass(2)
    for cp in pending_out:
        if cp is not None:
            cp.wait()
    acc_v[...] = acc
    pltpu.async_copy(acc_v, part_hbm.at[w], sem).wait()


def _apply_stage(density_grid, rloc, r01, r23, hist):
    k = pl.kernel(
        _apply_body,
        out_type=[
            jax.ShapeDtypeStruct((_SCENES, _CELLS), jnp.float32),
            jax.ShapeDtypeStruct((_NW, 16), jnp.float32),
        ],
        mesh=_SC_MESH(),
        compiler_params=_SC_PARAMS,
        scratch_types=[
            pltpu.VMEM((_BCELLS,), jnp.float32),
            pltpu.VMEM((_BCELLS,), jnp.float32),
            pltpu.VMEM((_BCELLS,), jnp.int32),
            pltpu.VMEM((_NW * _SLOT,), jnp.int32),
            pltpu.VMEM((_NW * _SLOT,), jnp.int32),
            pltpu.VMEM((_NW * _SLOT,), jnp.int32),
            pltpu.VMEM((_NW, _NB), jnp.int32),
            pltpu.SMEM((2 * _NW,), jnp.int32),
            pltpu.VMEM((16,), jnp.float32),
            pltpu.SemaphoreType.DMA,
        ],
    )
    return k(density_grid, rloc, r01, r23, hist)


def kernel(density_grid, code, W1, b1, Wc, W2, b2, coords):
    idx, s01, s23 = _sigma_stage(coords, code, W1, b1, Wc, W2, b2)
    rloc, r01, r23, hist = _route_stage(idx, s01, s23)
    new_grid, partials = _apply_stage(density_grid, rloc, r01, r23, hist)
    mean_density = jnp.sum(partials) / jnp.float32(_SCENES * _CELLS)
    return new_grid, mean_density
